# Initial kernel scaffold; baseline (speedup 1.0000x reference)
import jax
import jax.numpy as jnp
from jax.experimental import pallas as pl


def kernel(x, edge_index, W1, b1, W2, b2, temp):
    raise NotImplementedError("write your pallas kernel here")


# speedup vs baseline: 13.0811x; 13.0811x over previous
import functools
import math

import numpy as np
import jax
import jax.numpy as jnp
from jax import lax
from jax.experimental import pallas as pl
from jax.experimental.pallas import tpu as pltpu
from jax.experimental.pallas import tpu_sc as plsc

N = 50000
E = 800000
F = 64
HF = 32
KTEMP = 10
N_POLY = 10
A_COEF = 1.0
B_COEF = 1.0

BLK = 128
NB = E // BLK
BPT = 392
NB_PAD = 16 * BPT
GRP = 4
NGRP = BPT // GRP
DUMP = N
N_ACC = 50048
RPT = N_ACC // 16
N_DEG = 50176
DPT = N_DEG // 16
RZ = 391
NZQ = RPT // RZ
RBLK = 5000
GRID_R = N // RBLK


def _cn(n, x, lam):
    if n == 0:
        return 1.0
    if n == 1:
        return x - lam
    return (x - n - lam + 1) * _cn(n - 1, x, lam) - (n - 1) * lam * _cn(n - 2, x, lam)


_MAT = np.zeros((KTEMP + 1, N_POLY), np.float64)
_MAT[0, 0] = 1.0
for _i in range(1, KTEMP + 1):
    for _j in range(N_POLY):
        _MAT[_i, _j] = _cn(_j, float(_i), A_COEF) / math.factorial(_j)


def _prep_body(ei_ref, gidx_ref, hist_ref):
    i = pl.program_id(0)
    r = lax.broadcasted_iota(jnp.int32, (BLK, BLK), 0)
    valid = (i * BLK + r) < NB
    row = ei_ref[0]
    col = ei_ref[1]
    rowm = jnp.where(valid, row, 0)
    gidx_ref[0] = rowm
    gidx_ref[1] = rowm + N
    hist_ref[0] = jnp.where(valid, row, DUMP)
    hist_ref[1] = jnp.where(valid, col, DUMP)


def _prep(ei3):
    return pl.pallas_call(
        _prep_body,
        grid=(NB_PAD // BLK,),
        in_specs=[pl.BlockSpec((2, BLK, BLK), lambda i: (0, i, 0))],
        out_specs=[
            pl.BlockSpec((2, BLK, BLK), lambda i: (0, i, 0)),
            pl.BlockSpec((2, BLK, BLK), lambda i: (0, i, 0)),
        ],
        out_shape=[
            jax.ShapeDtypeStruct((2, NB_PAD, BLK), jnp.int32),
            jax.ShapeDtypeStruct((2, NB_PAD, BLK), jnp.int32),
        ],
    )(ei3)


_MESH = plsc.VectorSubcoreMesh(core_axis_name="c", subcore_axis_name="s")


@functools.partial(
    pl.kernel,
    out_type=jax.ShapeDtypeStruct((2 * N_DEG,), jnp.float32),
    mesh=_MESH,
    scratch_types=[
        pltpu.VMEM((GRP, BLK), jnp.int32),
        pltpu.VMEM((BLK,), jnp.float32),
        pltpu.VMEM((DPT,), jnp.float32),
        pltpu.VMEM_SHARED((N_DEG,), jnp.float32),
        pltpu.SemaphoreType.DMA,
    ],
    compiler_params=pltpu.CompilerParams(use_tc_tiling_on_sc=False),
)
def _deg_kernel(hist_hbm, deg_hbm, sidx, ones, zbuf, acc, sem_s):
    cid = lax.axis_index("c")
    sid = lax.axis_index("s")

    def fill1(i, _):
        ones[pl.ds(i * 16, 16)] = jnp.full((16,), 1.0, jnp.float32)
        return 0

    lax.fori_loop(0, BLK // 16, fill1, 0)

    def fill0(i, _):
        zbuf[pl.ds(i * 16, 16)] = jnp.zeros((16,), jnp.float32)
        return 0

    lax.fori_loop(0, DPT // 16, fill0, 0)
    pltpu.sync_copy(zbuf, acc.at[pl.ds(sid * DPT, DPT)])
    plsc.subcore_barrier()

    base_blk = sid * BPT

    def grp_body(g, _):
        b0 = base_blk + g * GRP
        pltpu.sync_copy(hist_hbm.at[cid, pl.ds(b0, GRP), :], sidx)
        descs = [
            pltpu.async_copy(ones, acc.at[sidx.at[j]], sem_s, add=True)
            for j in range(GRP)
        ]
        for d in descs:
            d.wait()
        return 0

    lax.fori_loop(0, NGRP, grp_body, 0)
    plsc.subcore_barrier()
    pltpu.sync_copy(acc.at[pl.ds(sid * DPT, DPT)], zbuf)
    pltpu.sync_copy(zbuf, deg_hbm.at[pl.ds(cid * N_DEG + sid * DPT, DPT)])


@functools.partial(
    pl.kernel,
    out_type=jax.ShapeDtypeStruct((2 * N_ACC, HF), jnp.float32),
    mesh=_MESH,
    scratch_types=[
        pltpu.VMEM((GRP, BLK), jnp.int32),
        pltpu.VMEM((GRP, BLK), jnp.int32),
        pltpu.VMEM((GRP, BLK, HF), jnp.float32),
        pltpu.VMEM((RZ, HF), jnp.float32),
        pltpu.VMEM_SHARED((N_ACC, HF), jnp.float32),
        pltpu.SemaphoreType.DMA,
        pltpu.SemaphoreType.DMA,
    ],
    compiler_params=pltpu.CompilerParams(use_tc_tiling_on_sc=False),
)
def _spmv_kernel(y_hbm, gidx_hbm, hist_hbm, s_hbm, gidx, sidx, rows, zstage, acc,
                 sem_g, sem_s):
    cid = lax.axis_index("c")
    sid = lax.axis_index("s")

    def fill0(i, _):
        zstage[i, pl.ds(0, 16)] = jnp.zeros((16,), jnp.float32)
        zstage[i, pl.ds(16, 16)] = jnp.zeros((16,), jnp.float32)
        return 0

    lax.fori_loop(0, RZ, fill0, 0)
    for q in range(NZQ):
        pltpu.sync_copy(zstage, acc.at[pl.ds(sid * RPT + q * RZ, RZ), :])
    plsc.subcore_barrier()

    base_blk = sid * BPT

    def grp_body(g, _):
        b0 = base_blk + g * GRP
        pltpu.sync_copy(gidx_hbm.at[cid, pl.ds(b0, GRP), :], gidx)
        pltpu.sync_copy(hist_hbm.at[1, pl.ds(b0, GRP), :], sidx)
        gd = [
            pltpu.async_copy(y_hbm.at[gidx.at[j]], rows.at[j], sem_g)
            for j in range(GRP)
        ]
        for d in gd:
            d.wait()
        sd = [
            pltpu.async_copy(rows.at[j], acc.at[sidx.at[j]], sem_s, add=True)
            for j in range(GRP)
        ]
        for d in sd:
            d.wait()
        return 0

    lax.fori_loop(0, NGRP, grp_body, 0)
    plsc.subcore_barrier()
    for q in range(NZQ):
        pltpu.sync_copy(acc.at[pl.ds(sid * RPT + q * RZ, RZ), :], zstage)
        pltpu.sync_copy(
            zstage, s_hbm.at[pl.ds(cid * N_ACC + sid * RPT + q * RZ, RZ), :])


def _mlp_body(x_ref, w1_ref, b1_ref, w2_ref, b2_ref, d1_ref,
              xx_ref, y_ref):
    xb = x_ref[...]
    h1 = jnp.maximum(
        jnp.dot(xb.astype(jnp.bfloat16), w1_ref[...].astype(jnp.bfloat16),
                preferred_element_type=jnp.float32) + b1_ref[...], 0.0)
    h = jnp.dot(h1.astype(jnp.bfloat16), w2_ref[...].astype(jnp.bfloat16),
                preferred_element_type=jnp.float32) + b2_ref[...]
    d1 = d1_ref[...]
    xx_ref[...] = h
    y = d1 * h
    y_ref[0] = y[:, :HF]
    y_ref[1] = y[:, HF:]


def _mlp(x, W1, b1, W2, b2, deg1):
    return pl.pallas_call(
        _mlp_body,
        grid=(GRID_R,),
        in_specs=[
            pl.BlockSpec((RBLK, 128), lambda i: (i, 0)),
            pl.BlockSpec((128, F), lambda i: (0, 0)),
            pl.BlockSpec((1, F), lambda i: (0, 0)),
            pl.BlockSpec((F, F), lambda i: (0, 0)),
            pl.BlockSpec((1, F), lambda i: (0, 0)),
            pl.BlockSpec((RBLK, 1), lambda i: (i, 0)),
        ],
        out_specs=[
            pl.BlockSpec((RBLK, F), lambda i: (i, 0)),
            pl.BlockSpec((2, RBLK, HF), lambda i: (0, i, 0)),
        ],
        out_shape=[
            jax.ShapeDtypeStruct((N, F), jnp.float32),
            jax.ShapeDtypeStruct((2, N, HF), jnp.float32),
        ],
    )(x, W1, b1, W2, b2, deg1)


def _dense_body(s_ref, xx_ref, d1_ref, d2_ref, xxn_ref, yn_ref):
    s_full = jnp.concatenate([s_ref[0], s_ref[1]], axis=1)
    d1 = d1_ref[...]
    d2 = d2_ref[...]
    dd = d1 * d2 - B_COEF
    xx = xx_ref[...]
    xn = d2 * s_full + dd * xx
    xxn_ref[...] = xn
    y = d1 * xn
    yn_ref[0] = y[:, :HF]
    yn_ref[1] = y[:, HF:]


def _dense(s3, xx, deg1, deg2):
    return pl.pallas_call(
        _dense_body,
        grid=(GRID_R,),
        in_specs=[
            pl.BlockSpec((2, RBLK, HF), lambda i: (0, i, 0)),
            pl.BlockSpec((RBLK, F), lambda i: (i, 0)),
            pl.BlockSpec((RBLK, 1), lambda i: (i, 0)),
            pl.BlockSpec((RBLK, 1), lambda i: (i, 0)),
        ],
        out_specs=[
            pl.BlockSpec((RBLK, F), lambda i: (i, 0)),
            pl.BlockSpec((2, RBLK, HF), lambda i: (0, i, 0)),
        ],
        out_shape=[
            jax.ShapeDtypeStruct((N, F), jnp.float32),
            jax.ShapeDtypeStruct((2, N, HF), jnp.float32),
        ],
    )(s3, xx, deg1, deg2)


def _comb_body(*refs):
    t_refs = refs[:N_POLY]
    temp_ref = refs[N_POLY]
    out_ref = refs[N_POLY + 1]
    T = [r[...] for r in t_refs]
    out = T[0] * temp_ref[0, 0]
    for i in range(1, KTEMP + 1):
        out1 = T[0] * float(np.float32(_MAT[i, 0]))
        for j in range(1, N_POLY):
            out1 = out1 + T[j] * float(np.float32(_MAT[i, j]))
        out = out + temp_ref[0, i] * out1
    out_ref[...] = out


CBLK = 1000


def _comb(tmps, temp2d):
    return pl.pallas_call(
        _comb_body,
        grid=(N // CBLK,),
        in_specs=[pl.BlockSpec((CBLK, F), lambda i: (i, 0))
                  for _ in range(N_POLY)]
        + [pl.BlockSpec((1, KTEMP + 1), lambda i: (0, 0))],
        out_specs=pl.BlockSpec((CBLK, F), lambda i: (i, 0)),
        out_shape=jax.ShapeDtypeStruct((N, F), jnp.float32),
    )(*tmps, temp2d)


def kernel(x, edge_index, W1, b1, W2, b2, temp):
    ei3 = edge_index.reshape(2, NB, BLK)
    gidx_all, hist_all = _prep(ei3)
    deg = _deg_kernel(hist_all).reshape(2, N_DEG)
    d1v = jnp.power(deg[0, :N].reshape(N, 1) + 1.0, -0.5)
    d2v = jnp.power(deg[1, :N].reshape(N, 1) + 1.0, -0.5)
    xx, y = _mlp(x, W1, b1.reshape(1, F), W2, b2.reshape(1, F), d1v)
    tmps = [xx]
    for k in range(1, N_POLY):
        s = _spmv_kernel(y.reshape(2 * N, HF), gidx_all, hist_all)
        s3 = s.reshape(2, N_ACC, HF)
        xx, y = _dense(s3, xx, d1v, d2v)
        tmps.append(xx)
    return _comb(tmps, temp.reshape(1, KTEMP + 1))


# speedup vs baseline: 18.4411x; 1.4098x over previous
import functools
import math

import numpy as np
import jax
import jax.numpy as jnp
from jax import lax
from jax.experimental import pallas as pl
from jax.experimental.pallas import tpu as pltpu
from jax.experimental.pallas import tpu_sc as plsc

N = 50000
E = 800000
F = 64
HF = 32
KTEMP = 10
N_POLY = 10
A_COEF = 1.0
B_COEF = 1.0

BLK = 128
NB = E // BLK
BPT = 392
NB_PAD = 16 * BPT
GRP = 4
NGRP = BPT // GRP
DUMP = N
N_ACC = 50048
RPT = N_ACC // 16
N_DEG = 50176
DPT = N_DEG // 16
RZ = 391
NZQ = RPT // RZ
RBLK = 5000
GRID_R = N // RBLK


def _cn(n, x, lam):
    if n == 0:
        return 1.0
    if n == 1:
        return x - lam
    return (x - n - lam + 1) * _cn(n - 1, x, lam) - (n - 1) * lam * _cn(n - 2, x, lam)


_MAT = np.zeros((KTEMP + 1, N_POLY), np.float64)
_MAT[0, 0] = 1.0
for _i in range(1, KTEMP + 1):
    for _j in range(N_POLY):
        _MAT[_i, _j] = _cn(_j, float(_i), A_COEF) / math.factorial(_j)


def _prep_body(ei_ref, gidx_ref, hist_ref):
    i = pl.program_id(0)
    r = lax.broadcasted_iota(jnp.int32, (BLK, BLK), 0)
    valid = (i * BLK + r) < NB
    row = ei_ref[0]
    col = ei_ref[1]
    rowm = jnp.where(valid, row, 0)
    gidx_ref[0] = rowm
    gidx_ref[1] = rowm + N
    hist_ref[0] = jnp.where(valid, row, DUMP)
    hist_ref[1] = jnp.where(valid, col, DUMP)


def _prep(ei3):
    return pl.pallas_call(
        _prep_body,
        grid=(NB_PAD // BLK,),
        in_specs=[pl.BlockSpec((2, BLK, BLK), lambda i: (0, i, 0))],
        out_specs=[
            pl.BlockSpec((2, BLK, BLK), lambda i: (0, i, 0)),
            pl.BlockSpec((2, BLK, BLK), lambda i: (0, i, 0)),
        ],
        out_shape=[
            jax.ShapeDtypeStruct((2, NB_PAD, BLK), jnp.int32),
            jax.ShapeDtypeStruct((2, NB_PAD, BLK), jnp.int32),
        ],
    )(ei3)


_MESH = plsc.VectorSubcoreMesh(core_axis_name="c", subcore_axis_name="s")


@functools.partial(
    pl.kernel,
    out_type=jax.ShapeDtypeStruct((2 * N_DEG,), jnp.float32),
    mesh=_MESH,
    scratch_types=[
        pltpu.VMEM((GRP, BLK), jnp.int32),
        pltpu.VMEM((BLK,), jnp.float32),
        pltpu.VMEM((DPT,), jnp.float32),
        pltpu.VMEM_SHARED((N_DEG,), jnp.float32),
        pltpu.SemaphoreType.DMA,
    ],
    compiler_params=pltpu.CompilerParams(use_tc_tiling_on_sc=False),
)
def _deg_kernel(hist_hbm, deg_hbm, sidx, ones, zbuf, acc, sem_s):
    cid = lax.axis_index("c")
    sid = lax.axis_index("s")

    def fill1(i, _):
        ones[pl.ds(i * 16, 16)] = jnp.full((16,), 1.0, jnp.float32)
        return 0

    lax.fori_loop(0, BLK // 16, fill1, 0)

    def fill0(i, _):
        zbuf[pl.ds(i * 16, 16)] = jnp.zeros((16,), jnp.float32)
        return 0

    lax.fori_loop(0, DPT // 16, fill0, 0)
    pltpu.sync_copy(zbuf, acc.at[pl.ds(sid * DPT, DPT)])
    plsc.subcore_barrier()

    base_blk = sid * BPT

    def grp_body(g, _):
        b0 = base_blk + g * GRP
        pltpu.sync_copy(hist_hbm.at[cid, pl.ds(b0, GRP), :], sidx)
        descs = [
            pltpu.async_copy(ones, acc.at[sidx.at[j]], sem_s, add=True)
            for j in range(GRP)
        ]
        for d in descs:
            d.wait()
        return 0

    lax.fori_loop(0, NGRP, grp_body, 0)
    plsc.subcore_barrier()
    pltpu.sync_copy(acc.at[pl.ds(sid * DPT, DPT)], zbuf)
    pltpu.sync_copy(zbuf, deg_hbm.at[pl.ds(cid * N_DEG + sid * DPT, DPT)])


IDXC = 28
NCHUNK = BPT // IDXC
KB = 4
DEPTH = 3


@functools.partial(
    pl.kernel,
    out_type=jax.ShapeDtypeStruct((2 * N_ACC, HF), jnp.float32),
    mesh=_MESH,
    scratch_types=[
        pltpu.VMEM((2, IDXC, BLK), jnp.int32),
        pltpu.VMEM((2, IDXC, BLK), jnp.int32),
        pltpu.VMEM((KB, BLK, HF), jnp.float32),
        pltpu.VMEM_SHARED((N_ACC, HF), jnp.float32),
        pltpu.SemaphoreType.DMA,
        pltpu.SemaphoreType.DMA,
        pltpu.SemaphoreType.DMA,
    ],
    compiler_params=pltpu.CompilerParams(use_tc_tiling_on_sc=False),
)
def _spmv_kernel(y_hbm, gidx_hbm, hist_hbm, z_hbm, s_hbm, gidx, sidx, rows, acc,
                 sem_i, sem_g, sem_s):
    cid = lax.axis_index("c")
    sid = lax.axis_index("s")
    pltpu.sync_copy(z_hbm, acc.at[pl.ds(sid * RPT, RPT), :])
    plsc.subcore_barrier()

    base_blk = sid * BPT
    pltpu.async_copy(gidx_hbm.at[cid, pl.ds(base_blk, IDXC), :], gidx.at[0],
                     sem_i)
    pltpu.async_copy(hist_hbm.at[1, pl.ds(base_blk, IDXC), :], sidx.at[0],
                     sem_i)

    def chunk_body(c, _):
        par = lax.rem(c, 2)
        b0 = base_blk + c * IDXC
        pltpu.make_async_copy(gidx_hbm.at[cid, pl.ds(b0, IDXC), :],
                              gidx.at[par], sem_i).wait()
        pltpu.make_async_copy(hist_hbm.at[1, pl.ds(b0, IDXC), :],
                              sidx.at[par], sem_i).wait()

        @pl.when(c + 1 < NCHUNK)
        def _prefetch():
            nxt = lax.rem(c + 1, 2)
            nb0 = base_blk + (c + 1) * IDXC
            pltpu.async_copy(gidx_hbm.at[cid, pl.ds(nb0, IDXC), :],
                             gidx.at[nxt], sem_i)
            pltpu.async_copy(hist_hbm.at[1, pl.ds(nb0, IDXC), :],
                             sidx.at[nxt], sem_i)

        for b in range(IDXC):
            if b >= KB:
                s = b - KB
                pltpu.make_async_copy(rows.at[s % KB],
                                      acc.at[sidx.at[par, s]], sem_s).wait()
            pltpu.async_copy(y_hbm.at[gidx.at[par, b]], rows.at[b % KB], sem_g)
            if b >= DEPTH:
                g = b - DEPTH
                pltpu.make_async_copy(y_hbm.at[gidx.at[par, g]],
                                      rows.at[g % KB], sem_g).wait()
                pltpu.async_copy(rows.at[g % KB], acc.at[sidx.at[par, g]],
                                 sem_s, add=True)
        for b in range(IDXC - DEPTH, IDXC):
            pltpu.make_async_copy(y_hbm.at[gidx.at[par, b]], rows.at[b % KB],
                                  sem_g).wait()
            pltpu.async_copy(rows.at[b % KB], acc.at[sidx.at[par, b]], sem_s,
                             add=True)
        for b in range(IDXC - KB, IDXC):
            pltpu.make_async_copy(rows.at[b % KB], acc.at[sidx.at[par, b]],
                                  sem_s).wait()
        return 0

    lax.fori_loop(0, NCHUNK, chunk_body, 0)
    plsc.subcore_barrier()
    pltpu.sync_copy(
        acc.at[pl.ds(sid * RPT, RPT), :],
        s_hbm.at[pl.ds(cid * N_ACC + sid * RPT, RPT), :])


def _mlp_body(x_ref, w1_ref, b1_ref, w2_ref, b2_ref, d1_ref,
              xx_ref, y_ref):
    xb = x_ref[...]
    h1 = jnp.maximum(
        jnp.dot(xb.astype(jnp.bfloat16), w1_ref[...].astype(jnp.bfloat16),
                preferred_element_type=jnp.float32) + b1_ref[...], 0.0)
    h = jnp.dot(h1.astype(jnp.bfloat16), w2_ref[...].astype(jnp.bfloat16),
                preferred_element_type=jnp.float32) + b2_ref[...]
    d1 = d1_ref[...]
    xx_ref[...] = h
    y = d1 * h
    y_ref[0] = y[:, :HF]
    y_ref[1] = y[:, HF:]


def _mlp(x, W1, b1, W2, b2, deg1):
    return pl.pallas_call(
        _mlp_body,
        grid=(GRID_R,),
        in_specs=[
            pl.BlockSpec((RBLK, 128), lambda i: (i, 0)),
            pl.BlockSpec((128, F), lambda i: (0, 0)),
            pl.BlockSpec((1, F), lambda i: (0, 0)),
            pl.BlockSpec((F, F), lambda i: (0, 0)),
            pl.BlockSpec((1, F), lambda i: (0, 0)),
            pl.BlockSpec((RBLK, 1), lambda i: (i, 0)),
        ],
        out_specs=[
            pl.BlockSpec((RBLK, F), lambda i: (i, 0)),
            pl.BlockSpec((2, RBLK, HF), lambda i: (0, i, 0)),
        ],
        out_shape=[
            jax.ShapeDtypeStruct((N, F), jnp.float32),
            jax.ShapeDtypeStruct((2, N, HF), jnp.float32),
        ],
    )(x, W1, b1, W2, b2, deg1)


def _dense_body(s_ref, xx_ref, d1_ref, d2_ref, xxn_ref, yn_ref):
    s_full = jnp.concatenate([s_ref[0], s_ref[1]], axis=1)
    d1 = d1_ref[...]
    d2 = d2_ref[...]
    dd = d1 * d2 - B_COEF
    xx = xx_ref[...]
    xn = d2 * s_full + dd * xx
    xxn_ref[...] = xn
    y = d1 * xn
    yn_ref[0] = y[:, :HF]
    yn_ref[1] = y[:, HF:]


def _dense(s3, xx, deg1, deg2):
    return pl.pallas_call(
        _dense_body,
        grid=(GRID_R,),
        in_specs=[
            pl.BlockSpec((2, RBLK, HF), lambda i: (0, i, 0)),
            pl.BlockSpec((RBLK, F), lambda i: (i, 0)),
            pl.BlockSpec((RBLK, 1), lambda i: (i, 0)),
            pl.BlockSpec((RBLK, 1), lambda i: (i, 0)),
        ],
        out_specs=[
            pl.BlockSpec((RBLK, F), lambda i: (i, 0)),
            pl.BlockSpec((2, RBLK, HF), lambda i: (0, i, 0)),
        ],
        out_shape=[
            jax.ShapeDtypeStruct((N, F), jnp.float32),
            jax.ShapeDtypeStruct((2, N, HF), jnp.float32),
        ],
    )(s3, xx, deg1, deg2)


def _comb_body(*refs):
    t_refs = refs[:N_POLY]
    temp_ref = refs[N_POLY]
    out_ref = refs[N_POLY + 1]
    T = [r[...] for r in t_refs]
    out = T[0] * temp_ref[0, 0]
    for i in range(1, KTEMP + 1):
        out1 = T[0] * float(np.float32(_MAT[i, 0]))
        for j in range(1, N_POLY):
            out1 = out1 + T[j] * float(np.float32(_MAT[i, j]))
        out = out + temp_ref[0, i] * out1
    out_ref[...] = out


CBLK = 1000


def _comb(tmps, temp2d):
    return pl.pallas_call(
        _comb_body,
        grid=(N // CBLK,),
        in_specs=[pl.BlockSpec((CBLK, F), lambda i: (i, 0))
                  for _ in range(N_POLY)]
        + [pl.BlockSpec((1, KTEMP + 1), lambda i: (0, 0))],
        out_specs=pl.BlockSpec((CBLK, F), lambda i: (i, 0)),
        out_shape=jax.ShapeDtypeStruct((N, F), jnp.float32),
    )(*tmps, temp2d)


def kernel(x, edge_index, W1, b1, W2, b2, temp):
    ei3 = edge_index.reshape(2, NB, BLK)
    gidx_all, hist_all = _prep(ei3)
    deg = _deg_kernel(hist_all).reshape(2, N_DEG)
    d1v = jnp.power(deg[0, :N].reshape(N, 1) + 1.0, -0.5)
    d2v = jnp.power(deg[1, :N].reshape(N, 1) + 1.0, -0.5)
    xx, y = _mlp(x, W1, b1.reshape(1, F), W2, b2.reshape(1, F), d1v)
    tmps = [xx]
    zeros = jnp.zeros((RPT, HF), jnp.float32)
    for k in range(1, N_POLY):
        s = _spmv_kernel(y.reshape(2 * N, HF), gidx_all, hist_all, zeros)
        s3 = s.reshape(2, N_ACC, HF)
        xx, y = _dense(s3, xx, d1v, d2v)
        tmps.append(xx)
    return _comb(tmps, temp.reshape(1, KTEMP + 1))


# speedup vs baseline: 18.7707x; 1.0179x over previous
import functools
import math

import numpy as np
import jax
import jax.numpy as jnp
from jax import lax
from jax.experimental import pallas as pl
from jax.experimental.pallas import tpu as pltpu
from jax.experimental.pallas import tpu_sc as plsc

N = 50000
E = 800000
F = 64
HF = 32
KTEMP = 10
N_POLY = 10
A_COEF = 1.0
B_COEF = 1.0

BLK = 128
NB = E // BLK
BPT = 392
NB_PAD = 16 * BPT
GRP = 4
NGRP = BPT // GRP
DUMP = N
N_ACC = 50048
RPT = N_ACC // 16
N_DEG = 50176
DPT = N_DEG // 16
RZ = 391
NZQ = RPT // RZ
RBLK = 5000
GRID_R = N // RBLK


def _cn(n, x, lam):
    if n == 0:
        return 1.0
    if n == 1:
        return x - lam
    return (x - n - lam + 1) * _cn(n - 1, x, lam) - (n - 1) * lam * _cn(n - 2, x, lam)


_MAT = np.zeros((KTEMP + 1, N_POLY), np.float64)
_MAT[0, 0] = 1.0
for _i in range(1, KTEMP + 1):
    for _j in range(N_POLY):
        _MAT[_i, _j] = _cn(_j, float(_i), A_COEF) / math.factorial(_j)


def _prep_body(ei_ref, gidx_ref, hist_ref):
    i = pl.program_id(0)
    r = lax.broadcasted_iota(jnp.int32, (BLK, BLK), 0)
    valid = (i * BLK + r) < NB
    row = ei_ref[0]
    col = ei_ref[1]
    rowm = jnp.where(valid, row, 0)
    gidx_ref[0] = rowm
    gidx_ref[1] = rowm + N
    hist_ref[0] = jnp.where(valid, row, DUMP)
    hist_ref[1] = jnp.where(valid, col, DUMP)


def _prep(ei3):
    return pl.pallas_call(
        _prep_body,
        grid=(NB_PAD // BLK,),
        in_specs=[pl.BlockSpec((2, BLK, BLK), lambda i: (0, i, 0))],
        out_specs=[
            pl.BlockSpec((2, BLK, BLK), lambda i: (0, i, 0)),
            pl.BlockSpec((2, BLK, BLK), lambda i: (0, i, 0)),
        ],
        out_shape=[
            jax.ShapeDtypeStruct((2, NB_PAD, BLK), jnp.int32),
            jax.ShapeDtypeStruct((2, NB_PAD, BLK), jnp.int32),
        ],
    )(ei3)


IDXC = 28
NCHUNK = BPT // IDXC

_MESH = plsc.VectorSubcoreMesh(core_axis_name="c", subcore_axis_name="s")


@functools.partial(
    pl.kernel,
    out_type=jax.ShapeDtypeStruct((2 * N_DEG,), jnp.float32),
    mesh=_MESH,
    scratch_types=[
        pltpu.VMEM((2, 28, BLK), jnp.int32),
        pltpu.VMEM((BLK,), jnp.float32),
        pltpu.VMEM((DPT,), jnp.float32),
        pltpu.VMEM_SHARED((N_DEG,), jnp.float32),
        pltpu.SemaphoreType.DMA,
        pltpu.SemaphoreType.DMA,
    ],
    compiler_params=pltpu.CompilerParams(use_tc_tiling_on_sc=False),
)
def _deg_kernel(hist_hbm, deg_hbm, sidx, ones, zbuf, acc, sem_i, sem_s):
    cid = lax.axis_index("c")
    sid = lax.axis_index("s")

    def fill1(i, _):
        ones[pl.ds(i * 16, 16)] = jnp.full((16,), 1.0, jnp.float32)
        return 0

    lax.fori_loop(0, BLK // 16, fill1, 0)

    def fill0(i, _):
        zbuf[pl.ds(i * 16, 16)] = jnp.zeros((16,), jnp.float32)
        return 0

    lax.fori_loop(0, DPT // 16, fill0, 0)
    pltpu.sync_copy(zbuf, acc.at[pl.ds(sid * DPT, DPT)])
    plsc.subcore_barrier()

    base_blk = sid * BPT
    pltpu.async_copy(hist_hbm.at[cid, pl.ds(base_blk, IDXC), :], sidx.at[0],
                     sem_i)

    def chunk_body(c, _):
        par = lax.rem(c, 2)
        b0 = base_blk + c * IDXC
        pltpu.make_async_copy(hist_hbm.at[cid, pl.ds(b0, IDXC), :],
                              sidx.at[par], sem_i).wait()

        @pl.when(c + 1 < NCHUNK)
        def _prefetch():
            nxt = lax.rem(c + 1, 2)
            nb0 = base_blk + (c + 1) * IDXC
            pltpu.async_copy(hist_hbm.at[cid, pl.ds(nb0, IDXC), :],
                             sidx.at[nxt], sem_i)

        for b in range(IDXC):
            pltpu.async_copy(ones, acc.at[sidx.at[par, b]], sem_s, add=True)
        for b in range(IDXC):
            pltpu.make_async_copy(ones, acc.at[sidx.at[par, b]],
                                  sem_s).wait()
        return 0

    lax.fori_loop(0, NCHUNK, chunk_body, 0)
    plsc.subcore_barrier()
    pltpu.sync_copy(acc.at[pl.ds(sid * DPT, DPT)], zbuf)
    pltpu.sync_copy(zbuf, deg_hbm.at[pl.ds(cid * N_DEG + sid * DPT, DPT)])


KB = 4
DEPTH = 3


@functools.partial(
    pl.kernel,
    out_type=jax.ShapeDtypeStruct((2 * N_ACC, HF), jnp.float32),
    mesh=_MESH,
    scratch_types=[
        pltpu.VMEM((2, IDXC, BLK), jnp.int32),
        pltpu.VMEM((2, IDXC, BLK), jnp.int32),
        pltpu.VMEM((KB, BLK, HF), jnp.float32),
        pltpu.VMEM_SHARED((N_ACC, HF), jnp.float32),
        pltpu.SemaphoreType.DMA,
        pltpu.SemaphoreType.DMA,
        pltpu.SemaphoreType.DMA,
    ],
    compiler_params=pltpu.CompilerParams(use_tc_tiling_on_sc=False),
)
def _spmv_kernel(y_hbm, gidx_hbm, hist_hbm, z_hbm, s_hbm, gidx, sidx, rows, acc,
                 sem_i, sem_g, sem_s):
    cid = lax.axis_index("c")
    sid = lax.axis_index("s")
    pltpu.sync_copy(z_hbm, acc.at[pl.ds(sid * RPT, RPT), :])
    plsc.subcore_barrier()

    base_blk = sid * BPT
    pltpu.async_copy(gidx_hbm.at[cid, pl.ds(base_blk, IDXC), :], gidx.at[0],
                     sem_i)
    pltpu.async_copy(hist_hbm.at[1, pl.ds(base_blk, IDXC), :], sidx.at[0],
                     sem_i)

    def chunk_body(c, _):
        par = lax.rem(c, 2)
        b0 = base_blk + c * IDXC
        pltpu.make_async_copy(gidx_hbm.at[cid, pl.ds(b0, IDXC), :],
                              gidx.at[par], sem_i).wait()
        pltpu.make_async_copy(hist_hbm.at[1, pl.ds(b0, IDXC), :],
                              sidx.at[par], sem_i).wait()

        @pl.when(c + 1 < NCHUNK)
        def _prefetch():
            nxt = lax.rem(c + 1, 2)
            nb0 = base_blk + (c + 1) * IDXC
            pltpu.async_copy(gidx_hbm.at[cid, pl.ds(nb0, IDXC), :],
                             gidx.at[nxt], sem_i)
            pltpu.async_copy(hist_hbm.at[1, pl.ds(nb0, IDXC), :],
                             sidx.at[nxt], sem_i)

        for b in range(IDXC):
            if b >= KB:
                s = b - KB
                pltpu.make_async_copy(rows.at[s % KB],
                                      acc.at[sidx.at[par, s]], sem_s).wait()
            pltpu.async_copy(y_hbm.at[gidx.at[par, b]], rows.at[b % KB], sem_g)
            if b >= DEPTH:
                g = b - DEPTH
                pltpu.make_async_copy(y_hbm.at[gidx.at[par, g]],
                                      rows.at[g % KB], sem_g).wait()
                pltpu.async_copy(rows.at[g % KB], acc.at[sidx.at[par, g]],
                                 sem_s, add=True)
        for b in range(IDXC - DEPTH, IDXC):
            pltpu.make_async_copy(y_hbm.at[gidx.at[par, b]], rows.at[b % KB],
                                  sem_g).wait()
            pltpu.async_copy(rows.at[b % KB], acc.at[sidx.at[par, b]], sem_s,
                             add=True)
        for b in range(IDXC - KB, IDXC):
            pltpu.make_async_copy(rows.at[b % KB], acc.at[sidx.at[par, b]],
                                  sem_s).wait()
        return 0

    lax.fori_loop(0, NCHUNK, chunk_body, 0)
    plsc.subcore_barrier()
    pltpu.sync_copy(
        acc.at[pl.ds(sid * RPT, RPT), :],
        s_hbm.at[pl.ds(cid * N_ACC + sid * RPT, RPT), :])


def _mlp_body(x_ref, w1_ref, b1_ref, w2_ref, b2_ref, d1_ref,
              xx_ref, y_ref):
    xb = x_ref[...]
    h1 = jnp.maximum(
        jnp.dot(xb.astype(jnp.bfloat16), w1_ref[...].astype(jnp.bfloat16),
                preferred_element_type=jnp.float32) + b1_ref[...], 0.0)
    h = jnp.dot(h1.astype(jnp.bfloat16), w2_ref[...].astype(jnp.bfloat16),
                preferred_element_type=jnp.float32) + b2_ref[...]
    d1 = d1_ref[...]
    xx_ref[...] = h
    y = d1 * h
    y_ref[0] = y[:, :HF]
    y_ref[1] = y[:, HF:]


def _mlp(x, W1, b1, W2, b2, deg1):
    return pl.pallas_call(
        _mlp_body,
        grid=(GRID_R,),
        in_specs=[
            pl.BlockSpec((RBLK, 128), lambda i: (i, 0)),
            pl.BlockSpec((128, F), lambda i: (0, 0)),
            pl.BlockSpec((1, F), lambda i: (0, 0)),
            pl.BlockSpec((F, F), lambda i: (0, 0)),
            pl.BlockSpec((1, F), lambda i: (0, 0)),
            pl.BlockSpec((RBLK, 1), lambda i: (i, 0)),
        ],
        out_specs=[
            pl.BlockSpec((RBLK, F), lambda i: (i, 0)),
            pl.BlockSpec((2, RBLK, HF), lambda i: (0, i, 0)),
        ],
        out_shape=[
            jax.ShapeDtypeStruct((N, F), jnp.float32),
            jax.ShapeDtypeStruct((2, N, HF), jnp.float32),
        ],
    )(x, W1, b1, W2, b2, deg1)


def _dense_body(s_ref, xx_ref, d1_ref, d2_ref, xxn_ref, yn_ref):
    s_full = jnp.concatenate([s_ref[0], s_ref[1]], axis=1)
    d1 = d1_ref[...]
    d2 = d2_ref[...]
    dd = d1 * d2 - B_COEF
    xx = xx_ref[...]
    xn = d2 * s_full + dd * xx
    xxn_ref[...] = xn
    y = d1 * xn
    yn_ref[0] = y[:, :HF]
    yn_ref[1] = y[:, HF:]


def _dense(s3, xx, deg1, deg2):
    return pl.pallas_call(
        _dense_body,
        grid=(GRID_R,),
        in_specs=[
            pl.BlockSpec((2, RBLK, HF), lambda i: (0, i, 0)),
            pl.BlockSpec((RBLK, F), lambda i: (i, 0)),
            pl.BlockSpec((RBLK, 1), lambda i: (i, 0)),
            pl.BlockSpec((RBLK, 1), lambda i: (i, 0)),
        ],
        out_specs=[
            pl.BlockSpec((RBLK, F), lambda i: (i, 0)),
            pl.BlockSpec((2, RBLK, HF), lambda i: (0, i, 0)),
        ],
        out_shape=[
            jax.ShapeDtypeStruct((N, F), jnp.float32),
            jax.ShapeDtypeStruct((2, N, HF), jnp.float32),
        ],
    )(s3, xx, deg1, deg2)


def _comb_body(*refs):
    t_refs = refs[:N_POLY]
    temp_ref = refs[N_POLY]
    out_ref = refs[N_POLY + 1]
    T = [r[...] for r in t_refs]
    out = T[0] * temp_ref[0, 0]
    for i in range(1, KTEMP + 1):
        out1 = T[0] * float(np.float32(_MAT[i, 0]))
        for j in range(1, N_POLY):
            out1 = out1 + T[j] * float(np.float32(_MAT[i, j]))
        out = out + temp_ref[0, i] * out1
    out_ref[...] = out


CBLK = 1000


def _comb(tmps, temp2d):
    return pl.pallas_call(
        _comb_body,
        grid=(N // CBLK,),
        in_specs=[pl.BlockSpec((CBLK, F), lambda i: (i, 0))
                  for _ in range(N_POLY)]
        + [pl.BlockSpec((1, KTEMP + 1), lambda i: (0, 0))],
        out_specs=pl.BlockSpec((CBLK, F), lambda i: (i, 0)),
        out_shape=jax.ShapeDtypeStruct((N, F), jnp.float32),
    )(*tmps, temp2d)


def kernel(x, edge_index, W1, b1, W2, b2, temp):
    ei3 = edge_index.reshape(2, NB, BLK)
    gidx_all, hist_all = _prep(ei3)
    deg = _deg_kernel(hist_all).reshape(2, N_DEG)
    d1v = jnp.power(deg[0, :N].reshape(N, 1) + 1.0, -0.5)
    d2v = jnp.power(deg[1, :N].reshape(N, 1) + 1.0, -0.5)
    xx, y = _mlp(x, W1, b1.reshape(1, F), W2, b2.reshape(1, F), d1v)
    tmps = [xx]
    zeros = jnp.zeros((RPT, HF), jnp.float32)
    for k in range(1, N_POLY):
        s = _spmv_kernel(y.reshape(2 * N, HF), gidx_all, hist_all, zeros)
        s3 = s.reshape(2, N_ACC, HF)
        xx, y = _dense(s3, xx, d1v, d2v)
        tmps.append(xx)
    return _comb(tmps, temp.reshape(1, KTEMP + 1))
